# baseline (speedup 1.0000x reference)
import functools

import jax
import jax.numpy as jnp
from jax import lax
from jax.experimental import pallas as pl
from jax.experimental.pallas import tpu as pltpu
from jax.experimental.pallas import tpu_sc as plsc

_NS = 16
_L = 16


def _route_kernel(x_ref, wg_ref, j_ref, *, T, E, CAP):
    x = x_ref[...]
    logits = jnp.dot(x, wg_ref[...], preferred_element_type=jnp.float32)
    m = jnp.max(logits, axis=1, keepdims=True)
    iota_e = jax.lax.broadcasted_iota(jnp.int32, (T, E), 1)
    a = jnp.min(jnp.where(logits == m, iota_e, E), axis=1, keepdims=True)
    oh = (iota_e == a).astype(jnp.float32)
    c = oh
    s = 1
    while s < T:
        shifted = jnp.concatenate(
            [jnp.zeros((s, E), jnp.float32), c[: T - s]], axis=0)
        c = c + shifted
        s *= 2
    slot = (jnp.sum(c * oh, axis=1, keepdims=True) - 1.0).astype(jnp.int32)
    kept = slot < CAP
    iota_t = jax.lax.broadcasted_iota(jnp.int32, (T, 1), 0)
    j_ref[...] = jnp.where(kept, a * CAP + slot,
                           E * CAP + (iota_t >> 4))


def _dispatch_body(j_hbm, tab_hbm, jv, vals, initv, sem, *, T, TBL):
    wid = lax.axis_index("s")
    tpw = T // _NS
    ipw = TBL // _NS
    base = wid * tpw
    pltpu.sync_copy(j_hbm.at[pl.ds(base, tpw)], jv)
    for kk in range(ipw // _L):
        initv[pl.ds(kk * _L, _L)] = jnp.full((_L,), T, jnp.int32)
    pltpu.sync_copy(initv, tab_hbm.at[pl.ds(wid * ipw, ipw)])
    for kk in range(tpw // _L):
        vals[pl.ds(kk * _L, _L)] = (
            lax.broadcasted_iota(jnp.int32, (_L,), 0) + (base + kk * _L))
    plsc.subcore_barrier()
    pltpu.async_copy(vals, tab_hbm.at[jv], sem).wait()


def _sc_dispatch(j_flat, *, T, TBL):
    mesh = plsc.VectorSubcoreMesh(
        core_axis_name="c", subcore_axis_name="s", num_cores=1)
    return pl.kernel(
        functools.partial(_dispatch_body, T=T, TBL=TBL),
        out_type=jax.ShapeDtypeStruct((TBL,), jnp.int32),
        mesh=mesh,
        scratch_types=[
            pltpu.VMEM((T // _NS,), jnp.int32),
            pltpu.VMEM((T // _NS,), jnp.int32),
            pltpu.VMEM((TBL // _NS,), jnp.int32),
            pltpu.SemaphoreType.DMA,
        ],
    )(j_flat)


def _expert_kernel(idx_ref, x_ref, w1_ref, b1_ref, w2_ref, b2_ref, out_ref,
                   *, T):
    e = pl.program_id(0)
    idxv = idx_ref[0]
    iota_t = jax.lax.broadcasted_iota(jnp.int32, (T, idxv.shape[1]), 0)
    p = (iota_t == idxv).astype(jnp.float32)
    xs = jax.lax.dot_general(
        p, x_ref[...], (((0,), (0,)), ((), ())),
        preferred_element_type=jnp.float32)
    h = jnp.maximum(
        jnp.dot(xs, w1_ref[0], preferred_element_type=jnp.float32) + b1_ref[0],
        0.0)
    ys = jnp.dot(h, w2_ref[0], preferred_element_type=jnp.float32) + b2_ref[0]
    contrib = jnp.dot(p, ys, preferred_element_type=jnp.float32)

    @pl.when(e == 0)
    def _():
        out_ref[...] = contrib

    @pl.when(e > 0)
    def _():
        out_ref[...] += contrib


def kernel(x, Wg, W1, b1, W2, b2):
    T, D = x.shape
    E = Wg.shape[1]
    H = W1.shape[2]
    CAP = max(1, (2 * T) // E)
    TBL = E * CAP + 8 * _NS * 4

    j2d = pl.pallas_call(
        functools.partial(_route_kernel, T=T, E=E, CAP=CAP),
        out_shape=jax.ShapeDtypeStruct((T, 1), jnp.int32),
    )(x, Wg)
    tab = _sc_dispatch(j2d.reshape(T), T=T, TBL=TBL)
    idx3 = tab[:E * CAP].reshape(E, 1, CAP)
    b1r = b1.reshape(E, 1, H)
    b2r = b2.reshape(E, 1, D)

    return pl.pallas_call(
        functools.partial(_expert_kernel, T=T),
        grid=(E,),
        in_specs=[
            pl.BlockSpec((1, 1, CAP), lambda e: (e, 0, 0)),
            pl.BlockSpec((T, D), lambda e: (0, 0)),
            pl.BlockSpec((1, D, H), lambda e: (e, 0, 0)),
            pl.BlockSpec((1, 1, H), lambda e: (e, 0, 0)),
            pl.BlockSpec((1, H, D), lambda e: (e, 0, 0)),
            pl.BlockSpec((1, 1, D), lambda e: (e, 0, 0)),
        ],
        out_specs=pl.BlockSpec((T, D), lambda e: (0, 0)),
        out_shape=jax.ShapeDtypeStruct((T, D), jnp.float32),
    )(idx3, x, W1, b1r, W2, b2r)


# speedup vs baseline: 1.0497x; 1.0497x over previous
import functools

import jax
import jax.numpy as jnp
from jax.experimental import pallas as pl
from jax.experimental.pallas import tpu as pltpu


def _moe_kernel(x_ref, wg_ref, w1_ref, b1_ref, w2_ref, b2_ref, out_ref,
                key_scr, *, T, E, CAP):
    e = pl.program_id(0)

    @pl.when(e == 0)
    def _route():
        x = x_ref[...]
        logits = jnp.dot(x, wg_ref[...], preferred_element_type=jnp.float32)
        m = jnp.max(logits, axis=1, keepdims=True)
        iota_e = jax.lax.broadcasted_iota(jnp.int32, (T, E), 1)
        a = jnp.min(jnp.where(logits == m, iota_e, E), axis=1, keepdims=True)
        oh = (iota_e == a).astype(jnp.float32)
        c = oh
        s = 1
        while s < T:
            shifted = jnp.concatenate(
                [jnp.zeros((s, E), jnp.float32), c[: T - s]], axis=0)
            c = c + shifted
            s *= 2
        slot = (jnp.sum(c * oh, axis=1, keepdims=True) - 1.0).astype(jnp.int32)
        key_scr[...] = a * 4096 + slot

    key = key_scr[...]
    iota_c = jax.lax.broadcasted_iota(jnp.int32, (T, CAP), 1)
    p = (key == e * 4096 + iota_c).astype(jnp.float32)
    xs = jax.lax.dot_general(
        p, x_ref[...], (((0,), (0,)), ((), ())),
        preferred_element_type=jnp.float32)
    h = jnp.maximum(
        jnp.dot(xs, w1_ref[0], preferred_element_type=jnp.float32) + b1_ref[0],
        0.0)
    ys = jnp.dot(h, w2_ref[0], preferred_element_type=jnp.float32) + b2_ref[0]
    contrib = jnp.dot(p, ys, preferred_element_type=jnp.float32)

    @pl.when(e == 0)
    def _init():
        out_ref[...] = contrib

    @pl.when(e > 0)
    def _acc():
        out_ref[...] += contrib


def kernel(x, Wg, W1, b1, W2, b2):
    T, D = x.shape
    E = Wg.shape[1]
    H = W1.shape[2]
    CAP = max(1, (2 * T) // E)
    b1r = b1.reshape(E, 1, H)
    b2r = b2.reshape(E, 1, D)

    return pl.pallas_call(
        functools.partial(_moe_kernel, T=T, E=E, CAP=CAP),
        grid=(E,),
        in_specs=[
            pl.BlockSpec((T, D), lambda e: (0, 0)),
            pl.BlockSpec((D, E), lambda e: (0, 0)),
            pl.BlockSpec((1, D, H), lambda e: (e, 0, 0)),
            pl.BlockSpec((1, 1, H), lambda e: (e, 0, 0)),
            pl.BlockSpec((1, H, D), lambda e: (e, 0, 0)),
            pl.BlockSpec((1, 1, D), lambda e: (e, 0, 0)),
        ],
        out_specs=pl.BlockSpec((T, D), lambda e: (0, 0)),
        out_shape=jax.ShapeDtypeStruct((T, D), jnp.float32),
        scratch_shapes=[pltpu.VMEM((T, 1), jnp.int32)],
    )(x, Wg, W1, b1r, W2, b2r)


# speedup vs baseline: 1.1554x; 1.1007x over previous
import functools

import jax
import jax.numpy as jnp
from jax.experimental import pallas as pl
from jax.experimental.pallas import tpu as pltpu


def _moe_kernel(x_ref, wg_ref, w1_ref, b1_ref, w2_ref, b2_ref, out_ref,
                idx_scr, *, T, E, CAP):
    e = pl.program_id(0)

    @pl.when(e == 0)
    def _route():
        x = x_ref[...]
        logits = jnp.dot(x, wg_ref[...], preferred_element_type=jnp.float32)
        m = jnp.max(logits, axis=1, keepdims=True)
        iota_e = jax.lax.broadcasted_iota(jnp.int32, (T, E), 1)
        a = jnp.min(jnp.where(logits == m, iota_e, E), axis=1, keepdims=True)
        oh = (iota_e == a).astype(jnp.float32)
        c = oh
        s = 1
        while s < T:
            shifted = jnp.concatenate(
                [jnp.zeros((s, E), jnp.float32), c[: T - s]], axis=0)
            c = c + shifted
            s *= 2
        slot = jnp.sum(c * oh, axis=1, keepdims=True) - 1.0
        iota_c = jax.lax.broadcasted_iota(jnp.int32, (T, CAP), 1)
        ohc = (iota_c == slot.astype(jnp.int32)).astype(jnp.float32)
        tvals = (jax.lax.broadcasted_iota(jnp.int32, (T, 1), 0)
                 + 1).astype(jnp.float32)
        tfe = jax.lax.dot_general(
            oh * tvals, ohc, (((0,), (0,)), ((), ())),
            preferred_element_type=jnp.float32)
        idx_scr[...] = jnp.where(tfe > 0.5, tfe.astype(jnp.int32) - 1, T)

    idxv = idx_scr[pl.ds(e, 1), :]
    iota_t = jax.lax.broadcasted_iota(jnp.int32, (T, CAP), 0)
    p = (iota_t == idxv).astype(jnp.float32)
    xs = jax.lax.dot_general(
        p, x_ref[...], (((0,), (0,)), ((), ())),
        preferred_element_type=jnp.float32)
    h = jnp.maximum(
        jnp.dot(xs, w1_ref[0], preferred_element_type=jnp.float32) + b1_ref[0],
        0.0)
    ys = jnp.dot(h, w2_ref[0], preferred_element_type=jnp.float32) + b2_ref[0]
    contrib = jnp.dot(p, ys, preferred_element_type=jnp.float32)

    @pl.when(e == 0)
    def _init():
        out_ref[...] = contrib

    @pl.when(e > 0)
    def _acc():
        out_ref[...] += contrib


def kernel(x, Wg, W1, b1, W2, b2):
    T, D = x.shape
    E = Wg.shape[1]
    H = W1.shape[2]
    CAP = max(1, (2 * T) // E)
    b1r = b1.reshape(E, 1, H)
    b2r = b2.reshape(E, 1, D)

    return pl.pallas_call(
        functools.partial(_moe_kernel, T=T, E=E, CAP=CAP),
        grid=(E,),
        in_specs=[
            pl.BlockSpec((T, D), lambda e: (0, 0)),
            pl.BlockSpec((D, E), lambda e: (0, 0)),
            pl.BlockSpec((1, D, H), lambda e: (e, 0, 0)),
            pl.BlockSpec((1, 1, H), lambda e: (e, 0, 0)),
            pl.BlockSpec((1, H, D), lambda e: (e, 0, 0)),
            pl.BlockSpec((1, 1, D), lambda e: (e, 0, 0)),
        ],
        out_specs=pl.BlockSpec((T, D), lambda e: (0, 0)),
        out_shape=jax.ShapeDtypeStruct((T, D), jnp.float32),
        scratch_shapes=[pltpu.VMEM((E, CAP), jnp.int32)],
    )(x, Wg, W1, b1r, W2, b2r)
